# Initial kernel scaffold; baseline (speedup 1.0000x reference)
import jax
import jax.numpy as jnp
from jax.experimental import pallas as pl


def kernel(z, edge_index):
    raise NotImplementedError("write your pallas kernel here")


# speedup vs baseline: 1.2095x; 1.2095x over previous
import functools

import jax
import jax.numpy as jnp
from jax import lax
from jax.experimental import pallas as pl
from jax.experimental.pallas import tpu as pltpu
from jax.experimental.pallas import tpu_sc as plsc

NW = 32
CH = 80
NCHUNK = 125
D = 128
NGRP = CH // 16


def _decoder_body(z_hbm, src_hbm, dst_hbm, out_hbm,
                  idx_s, idx_d, rows_s, rows_d, out_v, sem_s, sem_d):
    wid = lax.axis_index("s") * 2 + lax.axis_index("c")

    pltpu.sync_copy(src_hbm.at[wid], idx_s)
    pltpu.sync_copy(dst_hbm.at[wid], idx_d)

    def chunk_body(c, carry):
        cp_s = pltpu.async_copy(z_hbm.at[idx_s.at[c]], rows_s, sem_s)
        cp_d = pltpu.async_copy(z_hbm.at[idx_d.at[c]], rows_d, sem_d)
        cp_s.wait()
        cp_d.wait()

        for g in range(NGRP):
            row16 = g * 16 + lax.iota(jnp.int32, 16)

            def jbody(j, jcarry):
                ss, jv = jcarry
                a = plsc.load_gather(rows_s, [row16, jv])
                b = plsc.load_gather(rows_d, [row16, jv])
                d = a - b
                return ss + d * d, jv + 1

            ss, _ = lax.fori_loop(
                0, D - 1, jbody,
                (jnp.zeros((16,), jnp.float32), jnp.zeros((16,), jnp.int32)))

            mass = plsc.load_gather(
                rows_d, [row16, jnp.full((16,), D - 1, jnp.int32)])
            res = 1.0 / (1.0 + ss * jnp.exp(-mass))
            out_v[c, pl.ds(g * 16, 16)] = res
        return carry

    lax.fori_loop(0, NCHUNK, chunk_body, 0)

    pltpu.sync_copy(out_v, out_hbm.at[wid])


@jax.jit
def _decoder(z, src, dst):
    mesh = plsc.VectorSubcoreMesh(
        core_axis_name="c", subcore_axis_name="s", num_cores=2, num_subcores=16)
    f = pl.kernel(
        _decoder_body,
        out_type=jax.ShapeDtypeStruct((NW, NCHUNK, CH), jnp.float32),
        mesh=mesh,
        scratch_types=[
            pltpu.VMEM((NCHUNK, CH), jnp.int32),
            pltpu.VMEM((NCHUNK, CH), jnp.int32),
            pltpu.VMEM((CH, D), jnp.float32),
            pltpu.VMEM((CH, D), jnp.float32),
            pltpu.VMEM((NCHUNK, CH), jnp.float32),
            pltpu.SemaphoreType.DMA,
            pltpu.SemaphoreType.DMA,
        ],
        compiler_params=pltpu.CompilerParams(needs_layout_passes=False),
    )
    return f(z, src, dst)


def kernel(z, edge_index):
    src = edge_index[0].astype(jnp.int32).reshape(NW, NCHUNK, CH)
    dst = edge_index[1].astype(jnp.int32).reshape(NW, NCHUNK, CH)
    out = _decoder(z, src, dst)
    return out.reshape(-1, 1)


# speedup vs baseline: 1.3563x; 1.1213x over previous
import functools

import jax
import jax.numpy as jnp
from jax import lax
from jax.experimental import pallas as pl
from jax.experimental.pallas import tpu as pltpu
from jax.experimental.pallas import tpu_sc as plsc

NW = 32
CH = 80
NCHUNK = 125
D = 128
NGRP = CH // 16
UNROLL = 8


def _decoder_body(z_hbm, src_hbm, dst_hbm, out_hbm,
                  idx_s, idx_d, rows_s0, rows_d0, rows_s1, rows_d1, out_v,
                  sem_s0, sem_d0, sem_s1, sem_d1):
    wid = lax.axis_index("s") * 2 + lax.axis_index("c")

    pltpu.sync_copy(src_hbm.at[wid], idx_s)
    pltpu.sync_copy(dst_hbm.at[wid], idx_d)

    def start(c, rows_s, rows_d, sem_s, sem_d):
        pltpu.async_copy(z_hbm.at[idx_s.at[c]], rows_s, sem_s)
        pltpu.async_copy(z_hbm.at[idx_d.at[c]], rows_d, sem_d)

    def wait(c, rows_s, rows_d, sem_s, sem_d):
        pltpu.make_async_copy(z_hbm.at[idx_s.at[c]], rows_s, sem_s).wait()
        pltpu.make_async_copy(z_hbm.at[idx_d.at[c]], rows_d, sem_d).wait()

    def compute(c, rows_s, rows_d):
        for g in range(NGRP):
            row16 = g * 16 + lax.iota(jnp.int32, 16)

            def jbody(it, jcarry):
                ss, jv = jcarry
                for k in range(UNROLL):
                    col = jv + k
                    a = plsc.load_gather(rows_s, [row16, col])
                    b = plsc.load_gather(rows_d, [row16, col])
                    d = a - b
                    ss = ss + d * d
                return ss, jv + UNROLL

            ss, _ = lax.fori_loop(
                0, D // UNROLL, jbody,
                (jnp.zeros((16,), jnp.float32), jnp.zeros((16,), jnp.int32)))

            col127 = jnp.full((16,), D - 1, jnp.int32)
            a127 = plsc.load_gather(rows_s, [row16, col127])
            mass = plsc.load_gather(rows_d, [row16, col127])
            dm = a127 - mass
            ss = ss - dm * dm
            res = 1.0 / (1.0 + ss * jnp.exp(-mass))
            out_v[c, pl.ds(g * 16, 16)] = res

    start(0, rows_s0, rows_d0, sem_s0, sem_d0)

    def pair_body(i, carry):
        c0 = 2 * i
        start(c0 + 1, rows_s1, rows_d1, sem_s1, sem_d1)
        wait(c0, rows_s0, rows_d0, sem_s0, sem_d0)
        compute(c0, rows_s0, rows_d0)
        start(c0 + 2, rows_s0, rows_d0, sem_s0, sem_d0)
        wait(c0 + 1, rows_s1, rows_d1, sem_s1, sem_d1)
        compute(c0 + 1, rows_s1, rows_d1)
        return carry

    lax.fori_loop(0, (NCHUNK - 1) // 2, pair_body, 0)
    wait(NCHUNK - 1, rows_s0, rows_d0, sem_s0, sem_d0)
    compute(NCHUNK - 1, rows_s0, rows_d0)

    pltpu.sync_copy(out_v, out_hbm.at[wid])


@jax.jit
def _decoder(z, src, dst):
    mesh = plsc.VectorSubcoreMesh(
        core_axis_name="c", subcore_axis_name="s", num_cores=2, num_subcores=16)
    f = pl.kernel(
        _decoder_body,
        out_type=jax.ShapeDtypeStruct((NW, NCHUNK, CH), jnp.float32),
        mesh=mesh,
        scratch_types=[
            pltpu.VMEM((NCHUNK, CH), jnp.int32),
            pltpu.VMEM((NCHUNK, CH), jnp.int32),
            pltpu.VMEM((CH, D), jnp.float32),
            pltpu.VMEM((CH, D), jnp.float32),
            pltpu.VMEM((CH, D), jnp.float32),
            pltpu.VMEM((CH, D), jnp.float32),
            pltpu.VMEM((NCHUNK, CH), jnp.float32),
            pltpu.SemaphoreType.DMA,
            pltpu.SemaphoreType.DMA,
            pltpu.SemaphoreType.DMA,
            pltpu.SemaphoreType.DMA,
        ],
        compiler_params=pltpu.CompilerParams(needs_layout_passes=False),
    )
    return f(z, src, dst)


def kernel(z, edge_index):
    src = edge_index[0].astype(jnp.int32).reshape(NW, NCHUNK, CH)
    dst = edge_index[1].astype(jnp.int32).reshape(NW, NCHUNK, CH)
    out = _decoder(z, src, dst)
    return out.reshape(-1, 1)
